# baseline (speedup 1.0000x reference)
import jax
import jax.numpy as jnp
from jax.experimental import pallas as pl
from jax.experimental.pallas import tpu as pltpu

_BLK = 8000
_NB = 33
_EPS = 1e-10


def _body(x_ref, ex_ref, ey_ref, out_ref, acc_ref):
    i = pl.program_id(0)

    @pl.when(i == 0)
    def _init():
        acc_ref[...] = jnp.zeros_like(acc_ref)

    ex = ex_ref[...]
    ey = ey_ref[...]
    cx = 0.5 * (ex[:, :-1] + ex[:, 1:])
    cy = 0.5 * (ey[:, :-1] + ey[:, 1:])
    wx = ex[:, 1:2] - ex[:, 0:1]
    wy = ey[:, 1:2] - ey[:, 0:1]
    mx = -0.5 / (wx * wx)
    my = -0.5 / (wy * wy)

    a = x_ref[:, 0:1]
    b = x_ref[:, 1:2]
    tx = a - cx
    ty = b - cy
    kx = jnp.exp(tx * tx * mx)
    ky = jnp.exp(ty * ty * my)

    acc_ref[...] += jax.lax.dot_general(
        kx, ky, (((0,), (0,)), ((), ())), preferred_element_type=jnp.float32
    )

    @pl.when(i == pl.num_programs(0) - 1)
    def _fin():
        acc = acc_ref[...]
        out_ref[...] = acc / (jnp.sum(acc) + _EPS)


def kernel(x, bin_edges_x, bin_edges_y):
    n, d = x.shape
    ex = bin_edges_x.reshape(1, _NB)
    ey = bin_edges_y.reshape(1, _NB)
    return pl.pallas_call(
        _body,
        grid=(n // _BLK,),
        in_specs=[
            pl.BlockSpec((_BLK, d), lambda i: (i, 0)),
            pl.BlockSpec((1, _NB), lambda i: (0, 0)),
            pl.BlockSpec((1, _NB), lambda i: (0, 0)),
        ],
        out_specs=pl.BlockSpec((32, 32), lambda i: (0, 0)),
        out_shape=jax.ShapeDtypeStruct((32, 32), jnp.float32),
        scratch_shapes=[pltpu.VMEM((32, 32), jnp.float32)],
        compiler_params=pltpu.CompilerParams(
            dimension_semantics=("arbitrary",)
        ),
    )(x, ex, ey)


# speedup vs baseline: 6.4438x; 6.4438x over previous
import jax
import jax.numpy as jnp
from jax.experimental import pallas as pl
from jax.experimental.pallas import tpu as pltpu

_L = 8192
_EPS = 1e-10
_PAD_VAL = 1e9


def _body(xt_ref, q_ref, cx_ref, cy_ref, out_ref, acc_ref):
    i = pl.program_id(0)

    @pl.when(i == 0)
    def _init():
        acc_ref[...] = jnp.zeros_like(acc_ref)

    u = xt_ref[0:1, :] * q_ref[0:1, 0:1]
    v = xt_ref[1:2, :] * q_ref[1:2, 0:1]
    tx = u - cx_ref[...]
    ty = v - cy_ref[...]
    kx = jnp.exp(-(tx * tx))
    ky = jnp.exp(-(ty * ty))

    acc_ref[...] += jax.lax.dot_general(
        kx, ky, (((1,), (1,)), ((), ())), preferred_element_type=jnp.float32
    )

    @pl.when(i == pl.num_programs(0) - 1)
    def _fin():
        acc = acc_ref[...]
        out_ref[...] = acc / (jnp.sum(acc) + _EPS)


def kernel(x, bin_edges_x, bin_edges_y):
    n = x.shape[0]
    grid = (n + _L - 1) // _L
    npad = grid * _L - n

    xt = jnp.pad(
        x[:, :2].T, ((0, 0), (0, npad)), constant_values=_PAD_VAL
    )
    cx = 0.5 * (bin_edges_x[:-1] + bin_edges_x[1:])
    cy = 0.5 * (bin_edges_y[:-1] + bin_edges_y[1:])
    qx = 1.0 / ((bin_edges_x[1] - bin_edges_x[0]) * jnp.sqrt(2.0))
    qy = 1.0 / ((bin_edges_y[1] - bin_edges_y[0]) * jnp.sqrt(2.0))
    q = jnp.stack([qx, qy]).reshape(2, 1)
    cxs = (cx * qx).reshape(32, 1)
    cys = (cy * qy).reshape(32, 1)

    return pl.pallas_call(
        _body,
        grid=(grid,),
        in_specs=[
            pl.BlockSpec((2, _L), lambda i: (0, i)),
            pl.BlockSpec((2, 1), lambda i: (0, 0)),
            pl.BlockSpec((32, 1), lambda i: (0, 0)),
            pl.BlockSpec((32, 1), lambda i: (0, 0)),
        ],
        out_specs=pl.BlockSpec((32, 32), lambda i: (0, 0)),
        out_shape=jax.ShapeDtypeStruct((32, 32), jnp.float32),
        scratch_shapes=[pltpu.VMEM((32, 32), jnp.float32)],
        compiler_params=pltpu.CompilerParams(
            dimension_semantics=("arbitrary",)
        ),
    )(xt, q, cxs, cys)


# speedup vs baseline: 9.9510x; 1.5443x over previous
import jax
import jax.numpy as jnp
from jax.experimental import pallas as pl
from jax.experimental.pallas import tpu as pltpu

_L = 32768
_EPS = 1e-10
_PAD_VAL = 1e9


def _body(xt_ref, q_ref, cx_ref, cy_ref, out_ref, acc_ref):
    i = pl.program_id(0)

    @pl.when(i == 0)
    def _init():
        acc_ref[...] = jnp.zeros_like(acc_ref)

    u = xt_ref[0:1, :] * q_ref[0:1, 0:1]
    v = xt_ref[1:2, :] * q_ref[1:2, 0:1]
    tx = u - cx_ref[...]
    ty = v - cy_ref[...]
    kx = jnp.exp2(tx * (-tx))
    ky = jnp.exp2(ty * (-ty))

    acc_ref[...] += jax.lax.dot_general(
        kx, ky, (((1,), (1,)), ((), ())), preferred_element_type=jnp.float32
    )

    @pl.when(i == pl.num_programs(0) - 1)
    def _fin():
        acc = acc_ref[...]
        out_ref[...] = acc / (jnp.sum(acc) + _EPS)


def kernel(x, bin_edges_x, bin_edges_y):
    n = x.shape[0]
    grid = (n + _L - 1) // _L
    npad = grid * _L - n

    xt = jnp.pad(
        x[:, :2].T, ((0, 0), (0, npad)), constant_values=_PAD_VAL
    )
    cx = 0.5 * (bin_edges_x[:-1] + bin_edges_x[1:])
    cy = 0.5 * (bin_edges_y[:-1] + bin_edges_y[1:])
    scale = jnp.sqrt(jnp.log2(jnp.exp(1.0))) / jnp.sqrt(2.0)
    qx = scale / (bin_edges_x[1] - bin_edges_x[0])
    qy = scale / (bin_edges_y[1] - bin_edges_y[0])
    q = jnp.stack([qx, qy]).reshape(2, 1)
    cxs = (cx * qx).reshape(32, 1)
    cys = (cy * qy).reshape(32, 1)

    return pl.pallas_call(
        _body,
        grid=(grid,),
        in_specs=[
            pl.BlockSpec((2, _L), lambda i: (0, i)),
            pl.BlockSpec((2, 1), lambda i: (0, 0)),
            pl.BlockSpec((32, 1), lambda i: (0, 0)),
            pl.BlockSpec((32, 1), lambda i: (0, 0)),
        ],
        out_specs=pl.BlockSpec((32, 32), lambda i: (0, 0)),
        out_shape=jax.ShapeDtypeStruct((32, 32), jnp.float32),
        scratch_shapes=[pltpu.VMEM((32, 32), jnp.float32)],
        compiler_params=pltpu.CompilerParams(
            dimension_semantics=("arbitrary",)
        ),
    )(xt, q, cxs, cys)
